# baseline (speedup 1.0000x reference)
import functools

import jax
import jax.numpy as jnp
from jax import lax
from jax.experimental import pallas as pl
from jax.experimental.pallas import tpu as pltpu
from jax.experimental.pallas import tpu_sc as plsc

_N = 16777216
_NW = 32
_CHUNK = _N // _NW
_PIECE = 16384
_NP = _CHUNK // _PIECE
_NBUF = 7
_ZPIECE = 4096
_NZ = _CHUNK // _ZPIECE

_mesh = plsc.VectorSubcoreMesh(core_axis_name="c", subcore_axis_name="s")


@functools.partial(
    pl.kernel,
    mesh=_mesh,
    out_type=jax.ShapeDtypeStruct((2 * _N,), jnp.float32),
    scratch_types=(
        [pltpu.VMEM((_ZPIECE,), jnp.float32)]
        + [pltpu.VMEM((_PIECE,), jnp.float32)] * _NBUF
        + [pltpu.SemaphoreType.DMA]
        + [pltpu.SemaphoreType.DMA] * _NBUF
        + [pltpu.SemaphoreType.DMA]
    ),
)
def _sc_kernel(psi_hbm, out_hbm, zbuf, *scratch):
    bufs = list(scratch[:_NBUF])
    gsem = scratch[_NBUF]
    ssems = list(scratch[_NBUF + 1:_NBUF + 1 + _NBUF])
    zsem = scratch[_NBUF + 1 + _NBUF]
    wid = lax.axis_index("s") * 2 + lax.axis_index("c")
    base = wid * _CHUNK

    def gather(i, b):
        return pltpu.make_async_copy(
            psi_hbm.at[pl.ds(base + i * _PIECE, _PIECE)], bufs[b], gsem)

    def scatter(i, b):
        return pltpu.make_async_copy(
            bufs[b], out_hbm.at[pl.ds(base + i * _PIECE, _PIECE)], ssems[b])

    gathers = [gather(i, i % _NBUF) for i in range(_NBUF)]
    for g in gathers:
        g.start()

    def zstep(i, carry):
        for k in range(16):
            zbuf[pl.ds(i * 256 + k * 16, 16)] = jnp.zeros((16,), jnp.float32)
        return carry

    lax.fori_loop(0, _ZPIECE // 256, zstep, 0)

    zeros = [
        pltpu.make_async_copy(
            zbuf, out_hbm.at[pl.ds(_N + base + j * _ZPIECE, _ZPIECE)], zsem)
        for j in range(_NZ)
    ]
    for z in zeros:
        z.start()

    scatters = []
    for i in range(_NP):
        b = i % _NBUF
        gathers[i].wait()
        sc = scatter(i, b)
        sc.start()
        scatters.append(sc)
        ni = i + _NBUF
        if ni < _NP:
            sc.wait()
            g = gather(ni, b)
            g.start()
            gathers.append(g)

    for sc in scatters[_NP - _NBUF:]:
        sc.wait()
    for z in zeros:
        z.wait()


def kernel(psi):
    return _sc_kernel(psi)


# speedup vs baseline: 1.0160x; 1.0160x over previous
import functools

import jax
import jax.numpy as jnp
from jax import lax
from jax.experimental import pallas as pl
from jax.experimental.pallas import tpu as pltpu
from jax.experimental.pallas import tpu_sc as plsc

_N = 16777216
_NW = 32
_CHUNK = _N // _NW
_PIECE = 32768
_NP = _CHUNK // _PIECE
_NBUF = 3
_ZPIECE = 4096
_NZ = _CHUNK // _ZPIECE

_mesh = plsc.VectorSubcoreMesh(core_axis_name="c", subcore_axis_name="s")


@functools.partial(
    pl.kernel,
    mesh=_mesh,
    out_type=jax.ShapeDtypeStruct((2 * _N,), jnp.float32),
    scratch_types=(
        [pltpu.VMEM((_ZPIECE,), jnp.float32)]
        + [pltpu.VMEM((_PIECE,), jnp.float32)] * _NBUF
        + [pltpu.SemaphoreType.DMA]
        + [pltpu.SemaphoreType.DMA] * _NBUF
        + [pltpu.SemaphoreType.DMA]
    ),
)
def _sc_kernel(psi_hbm, out_hbm, zbuf, *scratch):
    bufs = list(scratch[:_NBUF])
    gsem = scratch[_NBUF]
    ssems = list(scratch[_NBUF + 1:_NBUF + 1 + _NBUF])
    zsem = scratch[_NBUF + 1 + _NBUF]
    wid = lax.axis_index("s") * 2 + lax.axis_index("c")
    base = wid * _CHUNK

    def gather(i, b):
        return pltpu.make_async_copy(
            psi_hbm.at[pl.ds(base + i * _PIECE, _PIECE)], bufs[b], gsem)

    def scatter(i, b):
        return pltpu.make_async_copy(
            bufs[b], out_hbm.at[pl.ds(base + i * _PIECE, _PIECE)], ssems[b])

    gathers = [gather(i, i % _NBUF) for i in range(_NBUF)]
    for g in gathers:
        g.start()

    def zstep(i, carry):
        for k in range(16):
            zbuf[pl.ds(i * 256 + k * 16, 16)] = jnp.zeros((16,), jnp.float32)
        return carry

    lax.fori_loop(0, _ZPIECE // 256, zstep, 0)

    zeros = [
        pltpu.make_async_copy(
            zbuf, out_hbm.at[pl.ds(_N + base + j * _ZPIECE, _ZPIECE)], zsem)
        for j in range(_NZ)
    ]
    for z in zeros:
        z.start()

    scatters = []
    for i in range(_NP):
        b = i % _NBUF
        gathers[i].wait()
        sc = scatter(i, b)
        sc.start()
        scatters.append(sc)
        ni = i + _NBUF
        if ni < _NP:
            sc.wait()
            g = gather(ni, b)
            g.start()
            gathers.append(g)

    for sc in scatters[_NP - _NBUF:]:
        sc.wait()
    for z in zeros:
        z.wait()


def kernel(psi):
    return _sc_kernel(psi)
